# Initial kernel scaffold; baseline (speedup 1.0000x reference)
import jax
import jax.numpy as jnp
from jax.experimental import pallas as pl


def kernel(cls_p3, reg_p3, cls_p4, reg_p4, cls_p5, reg_p5, gt_p3, gt_p4, gt_p5):
    raise NotImplementedError("write your pallas kernel here")


# speedup vs baseline: 5.3118x; 5.3118x over previous
import jax
import jax.numpy as jnp
from jax.experimental import pallas as pl
from jax.experimental.pallas import tpu as pltpu

_K = 16
_RATIO = 3.0
_MAXBITS = 0x7F7FFFFF


def _ce_pair(l0, l1, lbl_is1):
    m = jnp.maximum(l0, l1)
    lse = m + jnp.log(jnp.exp(l0 - m) + jnp.exp(l1 - m))
    return lse - jnp.where(lbl_is1, l1, l0)


def _smooth_l1(t, p):
    d = t - p
    ad = jnp.abs(d)
    return jnp.where(ad < 1.0, 0.5 * d * d, ad - 0.5)


def kernel(cls_p3, reg_p3, cls_p4, reg_p4, cls_p5, reg_p5, gt_p3, gt_p4, gt_p5):
    n_batch = cls_p3.shape[0]
    hws = []
    ins = []
    for c, r, g in ((cls_p3, reg_p3, gt_p3), (cls_p4, reg_p4, gt_p4),
                    (cls_p5, reg_p5, gt_p5)):
        hw = c.shape[2] * c.shape[3]
        hws.append(hw)
        ins += [c.reshape(n_batch, 4, hw), r.reshape(n_batch, 2 * _K, hw),
                g.reshape(n_batch, 3 + 2 * _K, hw)]
    ms = [float(n_batch * hw) for hw in hws]

    def body(cls3, reg3, gt3, cls4, reg4, gt4, cls5, reg5, gt5,
             out_ref, ce3, ce4, ce5, acc):
        n = pl.program_id(0)

        @pl.when(n == 0)
        def _init():
            for j in range(3):
                for i in range(8):
                    acc[j, i] = 0.0

        for j, (cls_r, reg_r, gt_r, ce_r) in enumerate(
                ((cls3, reg3, gt3, ce3), (cls4, reg4, gt4, ce4),
                 (cls5, reg5, gt5, ce5))):
            cls_b = cls_r[0]
            reg_b = reg_r[0]
            gt_b = gt_r[0]
            trm = gt_b[0:1]
            tclm = gt_b[1:2]
            trainm = gt_b[2:3]
            tgt = trm.astype(jnp.int32)
            tm = trainm.astype(jnp.int32)
            pos = (tgt * tm) > 0
            neg = ((1 - tgt) * tm) > 0
            ce_tr = _ce_pair(cls_b[0:1], cls_b[1:2], tgt == 1)
            acc[j, 0] += jnp.sum(pos.astype(jnp.float32))
            acc[j, 1] += jnp.sum(neg.astype(jnp.float32))
            acc[j, 2] += jnp.sum(jnp.where(pos, ce_tr, 0.0))
            bits = jax.lax.bitcast_convert_type(ce_tr, jnp.int32)
            ce_r[pl.ds(n, 1), :] = jnp.where(neg, bits, jnp.int32(-1))
            tr_train = (trainm * trm) > 0
            ce_tcl = _ce_pair(cls_b[2:3], cls_b[3:4],
                              tclm.astype(jnp.int32) == 1)
            acc[j, 3] += jnp.sum(jnp.where(tr_train, ce_tcl, 0.0))
            acc[j, 4] += jnp.sum(jnp.where(tr_train, 0.0, ce_tcl))
            acc[j, 5] += jnp.sum(tr_train.astype(jnp.float32))
            w = (trm + tclm) * 0.5
            sl1x = _smooth_l1(gt_b[3:3 + _K], reg_b[0:_K])
            sl1y = _smooth_l1(gt_b[3 + _K:3 + 2 * _K], reg_b[_K:2 * _K])
            acc[j, 6] += jnp.sum(jnp.where(tr_train, w * sl1x, 0.0))
            acc[j, 7] += jnp.sum(jnp.where(tr_train, w * sl1y, 0.0))

        @pl.when(n == n_batch - 1)
        def _final():
            ks = []
            nnegs = []
            for j in range(3):
                n_pos = acc[j, 0]
                neg_count = acc[j, 1]
                n_neg = jnp.where(
                    n_pos > 0,
                    jnp.minimum(neg_count, jnp.floor(_RATIO * n_pos)), 100.0)
                nnegs.append(n_neg)
                ks.append(jnp.minimum(n_neg, neg_count))

            b3 = ce3[...]
            b4 = ce4[...]
            b5 = ce5[...]

            def search_step(b, lo, hi, k):
                mid = lo + jax.lax.shift_right_logical(hi - lo + 1, 1)
                ge = jnp.sum((b >= mid).astype(jnp.float32)) >= k
                return jnp.where(ge, mid, lo), jnp.where(ge, hi, mid - 1)

            def loop_body(i, st):
                lo3, hi3, lo4, hi4, lo5, hi5 = st
                lo3, hi3 = search_step(b3, lo3, hi3, ks[0])
                lo4, hi4 = search_step(b4, lo4, hi4, ks[1])
                lo5, hi5 = search_step(b5, lo5, hi5, ks[2])
                return lo3, hi3, lo4, hi4, lo5, hi5

            z = jnp.int32(0)
            top = jnp.int32(_MAXBITS)
            st = jax.lax.fori_loop(0, 31, loop_body, (z, top, z, top, z, top))
            thresholds = (st[0], st[2], st[4])

            loss_text = 0.0
            loss_center = 0.0
            loss_rx = 0.0
            loss_ry = 0.0
            for j, b in enumerate((b3, b4, b5)):
                t = thresholds[j]
                gt_m = b > t
                cnt_gt = jnp.sum(gt_m.astype(jnp.float32))
                vals = jax.lax.bitcast_convert_type(b, jnp.float32)
                sum_gt = jnp.sum(jnp.where(gt_m, vals, 0.0))
                tval = jax.lax.bitcast_convert_type(t, jnp.float32)
                k = ks[j]
                loss_neg = jnp.where(k > 0, sum_gt + (k - cnt_gt) * tval, 0.0)
                n_pos = acc[j, 0]
                loss_pos = jnp.where(n_pos > 0, acc[j, 2], 0.0)
                loss_text += (loss_pos + loss_neg) / (n_pos + nnegs[j])
                p_count = acc[j, 5]
                ng_count = ms[j] - p_count
                loss_center += jnp.where(
                    p_count > 0,
                    acc[j, 3] / p_count + 0.5 * acc[j, 4] / ng_count, 0.0)
                loss_rx += jnp.where(p_count > 0,
                                     acc[j, 6] / (p_count * _K), 0.0)
                loss_ry += jnp.where(p_count > 0,
                                     acc[j, 7] / (p_count * _K), 0.0)
            out_ref[0] = loss_text
            out_ref[1] = loss_center
            out_ref[2] = loss_rx
            out_ref[3] = loss_ry

    specs = [pl.BlockSpec((1, a.shape[1], a.shape[2]), lambda n: (n, 0, 0))
             for a in ins]
    out = pl.pallas_call(
        body,
        grid=(n_batch,),
        in_specs=specs,
        out_specs=pl.BlockSpec(memory_space=pltpu.SMEM),
        out_shape=jax.ShapeDtypeStruct((4,), jnp.float32),
        scratch_shapes=[
            pltpu.VMEM((n_batch, hws[0]), jnp.int32),
            pltpu.VMEM((n_batch, hws[1]), jnp.int32),
            pltpu.VMEM((n_batch, hws[2]), jnp.int32),
            pltpu.SMEM((3, 8), jnp.float32),
        ],
    )(*ins)
    return (out[0], out[1], out[2], out[3])


# speedup vs baseline: 20.2169x; 3.8061x over previous
import jax
import jax.numpy as jnp
from jax.experimental import pallas as pl
from jax.experimental.pallas import tpu as pltpu

_K = 16
_RATIO = 3.0
_MAXBITS = 0x7F7FFFFF


def _ce_pair(l0, l1, lbl_is1):
    m = jnp.maximum(l0, l1)
    lse = m + jnp.log(jnp.exp(l0 - m) + jnp.exp(l1 - m))
    return lse - jnp.where(lbl_is1, l1, l0)


def _smooth_l1(t, p):
    d = t - p
    ad = jnp.abs(d)
    return jnp.where(ad < 1.0, 0.5 * d * d, ad - 0.5)


def kernel(cls_p3, reg_p3, cls_p4, reg_p4, cls_p5, reg_p5, gt_p3, gt_p4, gt_p5):
    n_batch = cls_p3.shape[0]
    ins = [cls_p3, reg_p3, gt_p3, cls_p4, reg_p4, gt_p4, cls_p5, reg_p5, gt_p5]
    hs = [cls_p3.shape[2], cls_p4.shape[2], cls_p5.shape[2]]
    ms = [float(n_batch * h * h) for h in hs]

    def body(cls3, reg3, gt3, cls4, reg4, gt4, cls5, reg5, gt5,
             out_ref, ce3, ce4, ce5, acc):
        n = pl.program_id(0)

        @pl.when(n == 0)
        def _init():
            for j in range(3):
                for i in range(8):
                    acc[j, i] = 0.0

        for j, (cls_r, reg_r, gt_r, ce_r) in enumerate(
                ((cls3, reg3, gt3, ce3), (cls4, reg4, gt4, ce4),
                 (cls5, reg5, gt5, ce5))):
            cls_b = cls_r[0]
            reg_b = reg_r[0]
            gt_b = gt_r[0]
            trm = gt_b[0:1]
            tclm = gt_b[1:2]
            trainm = gt_b[2:3]
            tgt = trm.astype(jnp.int32)
            tm = trainm.astype(jnp.int32)
            pos = (tgt * tm) > 0
            neg = ((1 - tgt) * tm) > 0
            ce_tr = _ce_pair(cls_b[0:1], cls_b[1:2], tgt == 1)
            acc[j, 0] += jnp.sum(pos.astype(jnp.float32))
            acc[j, 1] += jnp.sum(neg.astype(jnp.float32))
            acc[j, 2] += jnp.sum(jnp.where(pos, ce_tr, 0.0))
            bits = jax.lax.bitcast_convert_type(ce_tr, jnp.int32)
            ce_r[pl.ds(n, 1)] = jnp.where(neg, bits, jnp.int32(-1))
            tr_train = (trainm * trm) > 0
            ce_tcl = _ce_pair(cls_b[2:3], cls_b[3:4],
                              tclm.astype(jnp.int32) == 1)
            acc[j, 3] += jnp.sum(jnp.where(tr_train, ce_tcl, 0.0))
            acc[j, 4] += jnp.sum(jnp.where(tr_train, 0.0, ce_tcl))
            acc[j, 5] += jnp.sum(tr_train.astype(jnp.float32))
            w = (trm + tclm) * 0.5
            sl1x = _smooth_l1(gt_b[3:3 + _K], reg_b[0:_K])
            sl1y = _smooth_l1(gt_b[3 + _K:3 + 2 * _K], reg_b[_K:2 * _K])
            acc[j, 6] += jnp.sum(jnp.where(tr_train, w * sl1x, 0.0))
            acc[j, 7] += jnp.sum(jnp.where(tr_train, w * sl1y, 0.0))

        @pl.when(n == n_batch - 1)
        def _final():
            ks = []
            nnegs = []
            for j in range(3):
                n_pos = acc[j, 0]
                neg_count = acc[j, 1]
                n_neg = jnp.where(
                    n_pos > 0,
                    jnp.minimum(neg_count, jnp.floor(_RATIO * n_pos)), 100.0)
                nnegs.append(n_neg)
                ks.append(jnp.minimum(n_neg, neg_count))

            b3 = ce3[...]
            b4 = ce4[...]
            b5 = ce5[...]

            def search_step(b, lo, hi, k):
                mid = lo + jax.lax.shift_right_logical(hi - lo + 1, 1)
                ge = jnp.sum((b >= mid).astype(jnp.float32)) >= k
                return jnp.where(ge, mid, lo), jnp.where(ge, hi, mid - 1)

            def loop_body(i, st):
                lo3, hi3, lo4, hi4, lo5, hi5 = st
                lo3, hi3 = search_step(b3, lo3, hi3, ks[0])
                lo4, hi4 = search_step(b4, lo4, hi4, ks[1])
                lo5, hi5 = search_step(b5, lo5, hi5, ks[2])
                return lo3, hi3, lo4, hi4, lo5, hi5

            z = jnp.int32(0)
            top = jnp.int32(_MAXBITS)
            st = jax.lax.fori_loop(0, 31, loop_body, (z, top, z, top, z, top))
            thresholds = (st[0], st[2], st[4])

            loss_text = 0.0
            loss_center = 0.0
            loss_rx = 0.0
            loss_ry = 0.0
            for j, b in enumerate((b3, b4, b5)):
                t = thresholds[j]
                gt_m = b > t
                cnt_gt = jnp.sum(gt_m.astype(jnp.float32))
                vals = jax.lax.bitcast_convert_type(b, jnp.float32)
                sum_gt = jnp.sum(jnp.where(gt_m, vals, 0.0))
                tval = jax.lax.bitcast_convert_type(t, jnp.float32)
                k = ks[j]
                loss_neg = jnp.where(k > 0, sum_gt + (k - cnt_gt) * tval, 0.0)
                n_pos = acc[j, 0]
                loss_pos = jnp.where(n_pos > 0, acc[j, 2], 0.0)
                loss_text += (loss_pos + loss_neg) / (n_pos + nnegs[j])
                p_count = acc[j, 5]
                ng_count = ms[j] - p_count
                loss_center += jnp.where(
                    p_count > 0,
                    acc[j, 3] / p_count + 0.5 * acc[j, 4] / ng_count, 0.0)
                loss_rx += jnp.where(p_count > 0,
                                     acc[j, 6] / (p_count * _K), 0.0)
                loss_ry += jnp.where(p_count > 0,
                                     acc[j, 7] / (p_count * _K), 0.0)
            out_ref[0] = loss_text
            out_ref[1] = loss_center
            out_ref[2] = loss_rx
            out_ref[3] = loss_ry

    specs = [pl.BlockSpec((1,) + a.shape[1:], lambda n: (n, 0, 0, 0))
             for a in ins]
    out = pl.pallas_call(
        body,
        grid=(n_batch,),
        in_specs=specs,
        out_specs=pl.BlockSpec(memory_space=pltpu.SMEM),
        out_shape=jax.ShapeDtypeStruct((4,), jnp.float32),
        scratch_shapes=[
            pltpu.VMEM((n_batch, hs[0], hs[0]), jnp.int32),
            pltpu.VMEM((n_batch, hs[1], hs[1]), jnp.int32),
            pltpu.VMEM((n_batch, hs[2], hs[2]), jnp.int32),
            pltpu.SMEM((3, 8), jnp.float32),
        ],
    )(*ins)
    return (out[0], out[1], out[2], out[3])


# speedup vs baseline: 21.6844x; 1.0726x over previous
import jax
import jax.numpy as jnp
from jax.experimental import pallas as pl
from jax.experimental.pallas import tpu as pltpu

_K = 16
_RATIO = 3.0
_MAXBITS = 0x7F7FFFFF


def _ce_pair(l0, l1, lbl_is1):
    m = jnp.maximum(l0, l1)
    lse = m + jnp.log(jnp.exp(l0 - m) + jnp.exp(l1 - m))
    return lse - jnp.where(lbl_is1, l1, l0)


def _smooth_l1(t, p):
    ad = jnp.abs(t - p)
    m = jnp.minimum(ad, 1.0)
    return m * (ad - 0.5 * m)


def kernel(cls_p3, reg_p3, cls_p4, reg_p4, cls_p5, reg_p5, gt_p3, gt_p4, gt_p5):
    n_batch = cls_p3.shape[0]
    ins = [cls_p3, reg_p3, gt_p3, cls_p4, reg_p4, gt_p4, cls_p5, reg_p5, gt_p5]
    hs = [cls_p3.shape[2], cls_p4.shape[2], cls_p5.shape[2]]
    ms = [float(n_batch * h * h) for h in hs]

    def body(cls3, reg3, gt3, cls4, reg4, gt4, cls5, reg5, gt5,
             out_ref, ce3, ce4, ce5, acc3, acc4, acc5):
        n = pl.program_id(0)

        for j, (cls_r, reg_r, gt_r, ce_r, acc) in enumerate(
                ((cls3, reg3, gt3, ce3, acc3), (cls4, reg4, gt4, ce4, acc4),
                 (cls5, reg5, gt5, ce5, acc5))):
            trm = gt_r[0, 0]
            tclm = gt_r[0, 1]
            trainm = gt_r[0, 2]
            pos = (trm * trainm) > 0
            neg = (trainm - trm * trainm) > 0
            ce_tr = _ce_pair(cls_r[0, 0], cls_r[0, 1], trm > 0)
            bits = jax.lax.bitcast_convert_type(ce_tr, jnp.int32)
            ce_r[pl.ds(n, 1)] = jnp.where(neg, bits, jnp.int32(-1))[None]
            tr_train = pos
            ce_tcl = _ce_pair(cls_r[0, 2], cls_r[0, 3], tclm > 0)
            wmask = jnp.where(tr_train, (trm + tclm) * 0.5, 0.0)
            csx = jnp.sum(_smooth_l1(gt_r[0, 3:3 + _K], reg_r[0, 0:_K]),
                          axis=0)
            csy = jnp.sum(_smooth_l1(gt_r[0, 3 + _K:3 + 2 * _K],
                                     reg_r[0, _K:2 * _K]), axis=0)
            planes = (
                pos.astype(jnp.float32),
                neg.astype(jnp.float32),
                jnp.where(pos, ce_tr, 0.0),
                jnp.where(tr_train, ce_tcl, 0.0),
                jnp.where(tr_train, 0.0, ce_tcl),
                wmask * csx,
                wmask * csy,
            )

            @pl.when(n == 0)
            def _store():
                for i, q in enumerate(planes):
                    acc[i] = q

            @pl.when(n > 0)
            def _accum():
                for i, q in enumerate(planes):
                    acc[i] += q

        @pl.when(n == n_batch - 1)
        def _final():
            ks = []
            nnegs = []
            stats = []
            for acc in (acc3, acc4, acc5):
                n_pos = jnp.sum(acc[0])
                neg_count = jnp.sum(acc[1])
                stats.append((n_pos, neg_count, jnp.sum(acc[2]),
                              jnp.sum(acc[3]), jnp.sum(acc[4]),
                              jnp.sum(acc[5]), jnp.sum(acc[6])))
            for j in range(3):
                n_pos = stats[j][0]
                neg_count = stats[j][1]
                n_neg = jnp.where(
                    n_pos > 0,
                    jnp.minimum(neg_count, jnp.floor(_RATIO * n_pos)), 100.0)
                nnegs.append(n_neg)
                ks.append(jnp.minimum(n_neg, neg_count))

            b3 = ce3[...]
            b4 = ce4[...]
            b5 = ce5[...]

            def search_step(b, lo, hi, k):
                mid = lo + jax.lax.shift_right_logical(hi - lo + 1, 1)
                ge = jnp.sum((b >= mid).astype(jnp.float32)) >= k
                return jnp.where(ge, mid, lo), jnp.where(ge, hi, mid - 1)

            def loop_body(i, st):
                lo3, hi3, lo4, hi4, lo5, hi5 = st
                lo3, hi3 = search_step(b3, lo3, hi3, ks[0])
                lo4, hi4 = search_step(b4, lo4, hi4, ks[1])
                lo5, hi5 = search_step(b5, lo5, hi5, ks[2])
                return lo3, hi3, lo4, hi4, lo5, hi5

            z = jnp.int32(0)
            top = jnp.int32(_MAXBITS)
            st = jax.lax.fori_loop(0, 31, loop_body, (z, top, z, top, z, top))
            thresholds = (st[0], st[2], st[4])

            loss_text = 0.0
            loss_center = 0.0
            loss_rx = 0.0
            loss_ry = 0.0
            for j, b in enumerate((b3, b4, b5)):
                t = thresholds[j]
                gt_m = b > t
                cnt_gt = jnp.sum(gt_m.astype(jnp.float32))
                vals = jax.lax.bitcast_convert_type(b, jnp.float32)
                sum_gt = jnp.sum(jnp.where(gt_m, vals, 0.0))
                tval = jax.lax.bitcast_convert_type(t, jnp.float32)
                k = ks[j]
                loss_neg = jnp.where(k > 0, sum_gt + (k - cnt_gt) * tval, 0.0)
                n_pos = stats[j][0]
                loss_pos = jnp.where(n_pos > 0, stats[j][2], 0.0)
                loss_text += (loss_pos + loss_neg) / (n_pos + nnegs[j])
                p_count = n_pos
                ng_count = ms[j] - p_count
                loss_center += jnp.where(
                    p_count > 0,
                    stats[j][3] / p_count + 0.5 * stats[j][4] / ng_count, 0.0)
                loss_rx += jnp.where(p_count > 0,
                                     stats[j][5] / (p_count * _K), 0.0)
                loss_ry += jnp.where(p_count > 0,
                                     stats[j][6] / (p_count * _K), 0.0)
            out_ref[0] = loss_text
            out_ref[1] = loss_center
            out_ref[2] = loss_rx
            out_ref[3] = loss_ry

    specs = [pl.BlockSpec((1,) + a.shape[1:], lambda n: (n, 0, 0, 0))
             for a in ins]
    out = pl.pallas_call(
        body,
        grid=(n_batch,),
        in_specs=specs,
        out_specs=pl.BlockSpec(memory_space=pltpu.SMEM),
        out_shape=jax.ShapeDtypeStruct((4,), jnp.float32),
        scratch_shapes=[
            pltpu.VMEM((n_batch, hs[0], hs[0]), jnp.int32),
            pltpu.VMEM((n_batch, hs[1], hs[1]), jnp.int32),
            pltpu.VMEM((n_batch, hs[2], hs[2]), jnp.int32),
            pltpu.VMEM((7, hs[0], hs[0]), jnp.float32),
            pltpu.VMEM((7, hs[1], hs[1]), jnp.float32),
            pltpu.VMEM((7, hs[2], hs[2]), jnp.float32),
        ],
    )(*ins)
    return (out[0], out[1], out[2], out[3])


# speedup vs baseline: 22.5882x; 1.0417x over previous
import jax
import jax.numpy as jnp
from jax.experimental import pallas as pl
from jax.experimental.pallas import tpu as pltpu

_K = 16
_RATIO = 3.0
_MAXBITS = 0x7F7FFFFF


def _ce_pair(l0, l1, lbl_is1):
    m = jnp.maximum(l0, l1)
    lse = m + jnp.log(jnp.exp(l0 - m) + jnp.exp(l1 - m))
    return lse - jnp.where(lbl_is1, l1, l0)


def _smooth_l1(t, p):
    ad = jnp.abs(t - p)
    m = jnp.minimum(ad, 1.0)
    return m * (ad - 0.5 * m)


def kernel(cls_p3, reg_p3, cls_p4, reg_p4, cls_p5, reg_p5, gt_p3, gt_p4, gt_p5):
    n_batch = cls_p3.shape[0]
    ins = [cls_p3, reg_p3, gt_p3, cls_p4, reg_p4, gt_p4, cls_p5, reg_p5, gt_p5]
    hs = [cls_p3.shape[2], cls_p4.shape[2], cls_p5.shape[2]]
    ms = [float(n_batch * h * h) for h in hs]

    def body(cls3, reg3, gt3, cls4, reg4, gt4, cls5, reg5, gt5,
             out_ref, ce3, ce4, ce5, acc3, acc4, acc5):
        n = pl.program_id(0)

        for j, (cls_r, reg_r, gt_r, ce_r, acc) in enumerate(
                ((cls3, reg3, gt3, ce3, acc3), (cls4, reg4, gt4, ce4, acc4),
                 (cls5, reg5, gt5, ce5, acc5))):
            trm = gt_r[0, 0]
            tclm = gt_r[0, 1]
            trainm = gt_r[0, 2]
            pos = (trm * trainm) > 0
            neg = (trainm - trm * trainm) > 0
            ce_tr = _ce_pair(cls_r[0, 0], cls_r[0, 1], trm > 0)
            bits = jax.lax.bitcast_convert_type(ce_tr, jnp.int32)
            ce_r[pl.ds(n, 1)] = jnp.where(neg, bits, jnp.int32(-1))[None]
            tr_train = pos
            ce_tcl = _ce_pair(cls_r[0, 2], cls_r[0, 3], tclm > 0)
            wmask = jnp.where(tr_train, (trm + tclm) * 0.5, 0.0)
            csx = jnp.sum(_smooth_l1(gt_r[0, 3:3 + _K], reg_r[0, 0:_K]),
                          axis=0)
            csy = jnp.sum(_smooth_l1(gt_r[0, 3 + _K:3 + 2 * _K],
                                     reg_r[0, _K:2 * _K]), axis=0)
            ce_neg = jnp.where(neg, ce_tr, -1.0)
            ce_neg_inf = jnp.where(neg, ce_tr, jnp.inf)
            planes = (
                pos.astype(jnp.float32),
                neg.astype(jnp.float32),
                jnp.where(pos, ce_tr, 0.0),
                jnp.where(tr_train, ce_tcl, 0.0),
                jnp.where(tr_train, 0.0, ce_tcl),
                wmask * csx,
                wmask * csy,
            )

            @pl.when(n == 0)
            def _store():
                for i, q in enumerate(planes):
                    acc[i] = q
                acc[7] = ce_neg
                acc[8] = ce_neg_inf

            @pl.when(n > 0)
            def _accum():
                for i, q in enumerate(planes):
                    acc[i] += q
                acc[7] = jnp.maximum(acc[7], ce_neg)
                acc[8] = jnp.minimum(acc[8], ce_neg_inf)

        @pl.when(n == n_batch - 1)
        def _final():
            ks = []
            nnegs = []
            stats = []
            for acc in (acc3, acc4, acc5):
                n_pos = jnp.sum(acc[0])
                neg_count = jnp.sum(acc[1])
                stats.append((n_pos, neg_count, jnp.sum(acc[2]),
                              jnp.sum(acc[3]), jnp.sum(acc[4]),
                              jnp.sum(acc[5]), jnp.sum(acc[6])))
            for j in range(3):
                n_pos = stats[j][0]
                neg_count = stats[j][1]
                n_neg = jnp.where(
                    n_pos > 0,
                    jnp.minimum(neg_count, jnp.floor(_RATIO * n_pos)), 100.0)
                nnegs.append(n_neg)
                ks.append(jnp.minimum(n_neg, neg_count))

            b3 = ce3[...]
            b4 = ce4[...]
            b5 = ce5[...]

            bounds = []
            for acc in (acc3, acc4, acc5):
                mx = jax.lax.bitcast_convert_type(jnp.max(acc[7]), jnp.int32)
                mn = jax.lax.bitcast_convert_type(jnp.min(acc[8]), jnp.int32)
                bounds.append((jnp.minimum(jnp.maximum(mn, 0), _MAXBITS),
                               jnp.minimum(mx, _MAXBITS)))

            def search_step(b, lo, hi, k):
                mid = lo + jax.lax.shift_right_logical(hi - lo + 1, 1)
                ge = jnp.sum((b >= mid).astype(jnp.float32)) >= k
                return jnp.where(ge, mid, lo), jnp.where(ge, hi, mid - 1)

            def w_cond(st):
                lo3, hi3, lo4, hi4, lo5, hi5 = st
                return (lo3 < hi3) | (lo4 < hi4) | (lo5 < hi5)

            def w_body(st):
                lo3, hi3, lo4, hi4, lo5, hi5 = st
                lo3, hi3 = search_step(b3, lo3, hi3, ks[0])
                lo4, hi4 = search_step(b4, lo4, hi4, ks[1])
                lo5, hi5 = search_step(b5, lo5, hi5, ks[2])
                return lo3, hi3, lo4, hi4, lo5, hi5

            st = jax.lax.while_loop(
                w_cond, w_body,
                (bounds[0][0], bounds[0][1], bounds[1][0], bounds[1][1],
                 bounds[2][0], bounds[2][1]))
            thresholds = (st[0], st[2], st[4])

            loss_text = 0.0
            loss_center = 0.0
            loss_rx = 0.0
            loss_ry = 0.0
            for j, b in enumerate((b3, b4, b5)):
                t = thresholds[j]
                gt_m = b > t
                cnt_gt = jnp.sum(gt_m.astype(jnp.float32))
                vals = jax.lax.bitcast_convert_type(b, jnp.float32)
                sum_gt = jnp.sum(jnp.where(gt_m, vals, 0.0))
                tval = jax.lax.bitcast_convert_type(t, jnp.float32)
                k = ks[j]
                loss_neg = jnp.where(k > 0, sum_gt + (k - cnt_gt) * tval, 0.0)
                n_pos = stats[j][0]
                loss_pos = jnp.where(n_pos > 0, stats[j][2], 0.0)
                loss_text += (loss_pos + loss_neg) / (n_pos + nnegs[j])
                p_count = n_pos
                ng_count = ms[j] - p_count
                loss_center += jnp.where(
                    p_count > 0,
                    stats[j][3] / p_count + 0.5 * stats[j][4] / ng_count, 0.0)
                loss_rx += jnp.where(p_count > 0,
                                     stats[j][5] / (p_count * _K), 0.0)
                loss_ry += jnp.where(p_count > 0,
                                     stats[j][6] / (p_count * _K), 0.0)
            out_ref[0] = loss_text
            out_ref[1] = loss_center
            out_ref[2] = loss_rx
            out_ref[3] = loss_ry

    specs = [pl.BlockSpec((1,) + a.shape[1:], lambda n: (n, 0, 0, 0))
             for a in ins]
    out = pl.pallas_call(
        body,
        grid=(n_batch,),
        in_specs=specs,
        out_specs=pl.BlockSpec(memory_space=pltpu.SMEM),
        out_shape=jax.ShapeDtypeStruct((4,), jnp.float32),
        scratch_shapes=[
            pltpu.VMEM((n_batch, hs[0], hs[0]), jnp.int32),
            pltpu.VMEM((n_batch, hs[1], hs[1]), jnp.int32),
            pltpu.VMEM((n_batch, hs[2], hs[2]), jnp.int32),
            pltpu.VMEM((9, hs[0], hs[0]), jnp.float32),
            pltpu.VMEM((9, hs[1], hs[1]), jnp.float32),
            pltpu.VMEM((9, hs[2], hs[2]), jnp.float32),
        ],
    )(*ins)
    return (out[0], out[1], out[2], out[3])
